# Initial kernel scaffold; baseline (speedup 1.0000x reference)
import jax
import jax.numpy as jnp
from jax.experimental import pallas as pl


def kernel(x, edge_index1, edge_index2, pos_src, pos_dst, neg_src, neg_dst, W1, loopW1, b1, W2, loopW2, b2, pW1, pb1, pW2, pb2):
    raise NotImplementedError("write your pallas kernel here")


# speedup vs baseline: 3.1245x; 3.1245x over previous
import functools
import jax
import jax.numpy as jnp
from jax import lax
from jax.experimental import pallas as pl
from jax.experimental.pallas import tpu as pltpu
from jax.experimental.pallas import tpu_sc as plsc

N_NODES = 10000
F = 128
N_EDGES = 320000
N_PAIRS = 8192

NC = 2
NS = 16
NW = NC * NS

CHUNK = 128
EDGES_PER_W = 10240
NCH = EDGES_PER_W // CHUNK
E_PAD = EDGES_PER_W * NW
TOTAL_CH = E_PAD // CHUNK
ACC_ROWS = 10240
DUMMY_ROW = N_NODES

_ZERO16 = None


def _agg_body(h_hbm, src_hbm, dst_hbm, out_hbm, sidx, didx, rows0, rows1, acc, sem):
    c = lax.axis_index("c")
    s = lax.axis_index("s")
    w = c * NS + s

    zero16 = jnp.zeros((16,), jnp.float32)

    def zrow(r, _):
        for cb in range(8):
            rows0[r, pl.ds(cb * 16, 16)] = zero16
        return 0

    lax.fori_loop(0, CHUNK, zrow, 0)
    rows_per_tile = ACC_ROWS // NS
    for k in range(rows_per_tile // CHUNK):
        pltpu.sync_copy(rows0, acc.at[pl.ds(s * rows_per_tile + k * CHUNK, CHUNK)])

    plsc.subcore_barrier()

    half = NCH // 2
    for hf in range(2):
        pltpu.sync_copy(src_hbm.at[pl.ds(w * NCH + hf * half, half)], sidx)
        pltpu.sync_copy(dst_hbm.at[pl.ds(w * NCH + hf * half, half)], didx)

        def chunk_body(i, _):
            for b in range(2):
                j = i * 2 + b
                buf = rows0 if b == 0 else rows1
                pltpu.async_copy(h_hbm.at[sidx.at[j]], buf, sem).wait()
                pltpu.sync_copy(buf, acc.at[didx.at[j]], add=True)
            return 0

        lax.fori_loop(0, half // 2, chunk_body, 0)

    plsc.subcore_barrier()

    out_rows = ACC_ROWS // NS
    pltpu.sync_copy(acc.at[pl.ds(s * out_rows, out_rows)],
                    out_hbm.at[c, pl.ds(s * out_rows, out_rows)])


@jax.jit
def _sc_aggregate(h, src2d, dst2d):
    mesh = plsc.VectorSubcoreMesh(core_axis_name="c", subcore_axis_name="s",
                                  num_cores=NC, num_subcores=NS)
    return pl.kernel(
        _agg_body,
        out_type=jax.ShapeDtypeStruct((NC, ACC_ROWS, F), jnp.float32),
        mesh=mesh,
        scratch_types=[
            pltpu.VMEM((NCH // 2, CHUNK), jnp.int32),
            pltpu.VMEM((NCH // 2, CHUNK), jnp.int32),
            pltpu.VMEM((CHUNK, F), jnp.float32),
            pltpu.VMEM((CHUNK, F), jnp.float32),
            pltpu.VMEM_SHARED((ACC_ROWS, F), jnp.float32),
            pltpu.SemaphoreType.DMA,
        ],
    )(h, src2d, dst2d)


def _layer_tc_body(parts_ref, h_ref, wc_ref, lw_ref, b_ref, out_ref):
    asum = parts_ref[0] + parts_ref[1]
    acc = jnp.dot(asum, wc_ref[...], preferred_element_type=jnp.float32)
    acc += jnp.dot(h_ref[...], lw_ref[...], preferred_element_type=jnp.float32)
    acc += b_ref[...]
    out_ref[...] = jnp.maximum(acc, 0.0)


@jax.jit
def _tc_layer(parts, h, wc, lw, b2d):
    nb = 10
    rows = N_NODES // nb
    return pl.pallas_call(
        _layer_tc_body,
        grid=(nb,),
        in_specs=[
            pl.BlockSpec((NC, rows, F), lambda i: (0, i, 0)),
            pl.BlockSpec((rows, F), lambda i: (i, 0)),
            pl.BlockSpec((F, F), lambda i: (0, 0)),
            pl.BlockSpec((F, F), lambda i: (0, 0)),
            pl.BlockSpec((1, F), lambda i: (0, 0)),
        ],
        out_specs=pl.BlockSpec((rows, F), lambda i: (i, 0)),
        out_shape=jax.ShapeDtypeStruct((N_NODES, F), jnp.float32),
    )(parts, h, wc, lw, b2d)


def _pair_body(h_hbm, idx_hbm, out_hbm, idxv, bufa, bufb, sem):
    c = lax.axis_index("c")
    s = lax.axis_index("s")
    w = c * NS + s
    rows_per_w = N_PAIRS // CHUNK // NW

    pltpu.sync_copy(idx_hbm.at[w], idxv)

    for t in range(2):
        for k in range(rows_per_w):
            j = w * rows_per_w + k
            pltpu.async_copy(h_hbm.at[idxv.at[(2 * t) * 2 + k]], bufa, sem).wait()
            pltpu.async_copy(h_hbm.at[idxv.at[(2 * t + 1) * 2 + k]], bufb, sem).wait()

            def mrow(r, _):
                for cb in range(8):
                    sl = pl.ds(cb * 16, 16)
                    bufa[r, sl] = bufa[r, sl] * bufb[r, sl]
                return 0

            lax.fori_loop(0, CHUNK, mrow, 0)
            pltpu.sync_copy(bufa, out_hbm.at[t, pl.ds(j * CHUNK, CHUNK)])


@jax.jit
def _sc_pair_products(h, idx4):
    mesh = plsc.VectorSubcoreMesh(core_axis_name="c", subcore_axis_name="s",
                                  num_cores=NC, num_subcores=NS)
    rows_per_w = N_PAIRS // CHUNK // NW
    return pl.kernel(
        _pair_body,
        out_type=jax.ShapeDtypeStruct((2, N_PAIRS, F), jnp.float32),
        mesh=mesh,
        scratch_types=[
            pltpu.VMEM((4 * rows_per_w, CHUNK), jnp.int32),
            pltpu.VMEM((CHUNK, F), jnp.float32),
            pltpu.VMEM((CHUNK, F), jnp.float32),
            pltpu.SemaphoreType.DMA,
        ],
    )(h, idx4)


def _pred_body(x_ref, w1_ref, b1_ref, w2r_ref, b2_ref, out_ref):
    z = jnp.dot(x_ref[...], w1_ref[...], preferred_element_type=jnp.float32)
    z = jnp.maximum(z + b1_ref[...], 0.0)
    o = jnp.sum(z * w2r_ref[...], axis=1, keepdims=True) + b2_ref[...]
    out_ref[...] = o


@jax.jit
def _tc_predictor(x, pW1, pb1, pW2, pb2):
    m = 2 * N_PAIRS
    nb = 16
    rows = m // nb
    hid = pW1.shape[1]
    return pl.pallas_call(
        _pred_body,
        grid=(nb,),
        in_specs=[
            pl.BlockSpec((rows, F), lambda i: (i, 0)),
            pl.BlockSpec((F, hid), lambda i: (0, 0)),
            pl.BlockSpec((1, hid), lambda i: (0, 0)),
            pl.BlockSpec((1, hid), lambda i: (0, 0)),
            pl.BlockSpec((1, 1), lambda i: (0, 0)),
        ],
        out_specs=pl.BlockSpec((rows, 1), lambda i: (i, 0)),
        out_shape=jax.ShapeDtypeStruct((m, 1), jnp.float32),
    )(x, pW1, pb1, pW2, pb2)


def _pad_edges(edge_index):
    src = edge_index[0].astype(jnp.int32)
    dst = edge_index[1].astype(jnp.int32)
    pad = E_PAD - N_EDGES
    src = jnp.concatenate([src, jnp.zeros((pad,), jnp.int32)])
    dst = jnp.concatenate([dst, jnp.full((pad,), DUMMY_ROW, jnp.int32)])
    return src.reshape(TOTAL_CH, CHUNK), dst.reshape(TOTAL_CH, CHUNK)


def _block_diag(W):
    nb, bi, bo = W.shape
    eye = jnp.eye(nb, dtype=W.dtype)
    return (eye[:, None, :, None] * W[:, :, None, :]).reshape(nb * bi, nb * bo)


def kernel(x, edge_index1, edge_index2, pos_src, pos_dst, neg_src, neg_dst,
           W1, loopW1, b1, W2, loopW2, b2, pW1, pb1, pW2, pb2):
    src1, dst1 = _pad_edges(edge_index1)
    src2, dst2 = _pad_edges(edge_index2)

    wc1 = _block_diag(W1)
    wc2 = _block_diag(W2)

    parts1 = _sc_aggregate(x, src1, dst1)
    h1 = _tc_layer(parts1, x, wc1, loopW1, b1.reshape(1, F))

    parts2 = _sc_aggregate(h1, src2, dst2)
    h2 = _tc_layer(parts2, h1, wc2, loopW2, b2.reshape(1, F))

    idx4 = jnp.stack([pos_src, pos_dst, neg_src, neg_dst]).astype(jnp.int32)
    idx4 = idx4.reshape(4, NW, N_PAIRS // CHUNK // NW, CHUNK)
    idx4 = idx4.transpose(1, 0, 2, 3).reshape(NW, -1, CHUNK)
    prods = _sc_pair_products(h2, idx4)

    scores = _tc_predictor(prods.reshape(2 * N_PAIRS, F),
                           pW1, pb1.reshape(1, -1), pW2.reshape(1, -1),
                           pb2.reshape(1, 1))
    h_pos = scores[:N_PAIRS]
    h_neg = scores[N_PAIRS:]
    return (h_pos, h_neg, h2)


# speedup vs baseline: 3.3430x; 1.0699x over previous
import functools
import jax
import jax.numpy as jnp
from jax import lax
from jax.experimental import pallas as pl
from jax.experimental.pallas import tpu as pltpu
from jax.experimental.pallas import tpu_sc as plsc

N_NODES = 10000
F = 128
N_EDGES = 320000
N_PAIRS = 8192

NC = 2
NS = 16
NW = NC * NS

CHUNK = 128
EDGES_PER_W = 10240
NCH = EDGES_PER_W // CHUNK
E_PAD = EDGES_PER_W * NW
TOTAL_CH = E_PAD // CHUNK
ACC_ROWS = 10240
DUMMY_ROW = N_NODES

_ZERO16 = None


def _agg_body(h_hbm, src_hbm, dst_hbm, out_hbm, sidx, didx, rows0, rows1, acc,
              sg0, sg1, ss0, ss1):
    c = lax.axis_index("c")
    s = lax.axis_index("s")
    w = c * NS + s

    zero16 = jnp.zeros((16,), jnp.float32)

    def zrow(r, _):
        for cb in range(8):
            rows0[r, pl.ds(cb * 16, 16)] = zero16
        return 0

    lax.fori_loop(0, CHUNK, zrow, 0)
    rows_per_tile = ACC_ROWS // NS
    for k in range(rows_per_tile // CHUNK):
        pltpu.sync_copy(rows0, acc.at[pl.ds(s * rows_per_tile + k * CHUNK, CHUNK)])

    plsc.subcore_barrier()

    half = NCH // 2
    bufs = (rows0, rows1)
    gsem = (sg0, sg1)
    ssem = (ss0, ss1)

    def g_start(j, b):
        pltpu.async_copy(h_hbm.at[sidx.at[j]], bufs[b], gsem[b])

    def g_wait(j, b):
        pltpu.make_async_copy(h_hbm.at[sidx.at[j]], bufs[b], gsem[b]).wait()

    def s_start(j, b):
        pltpu.async_copy(bufs[b], acc.at[didx.at[j]], ssem[b], add=True)

    def s_wait(j, b):
        pltpu.make_async_copy(bufs[b], acc.at[didx.at[j]], ssem[b]).wait()

    for hf in range(2):
        pltpu.sync_copy(src_hbm.at[pl.ds(w * NCH + hf * half, half)], sidx)
        pltpu.sync_copy(dst_hbm.at[pl.ds(w * NCH + hf * half, half)], didx)

        g_start(0, 0)
        g_wait(0, 0)
        s_start(0, 0)
        g_start(1, 1)

        def chunk_body(i, _):
            for p in range(2):
                j = 2 * i + 1 + p
                b = (1 - p)
                g_wait(j, b)
                s_wait(j - 1, 1 - b)
                s_start(j, b)
                g_start(j + 1, 1 - b)
            return 0

        lax.fori_loop(0, (half - 2) // 2, chunk_body, 0)

        g_wait(half - 1, 1)
        s_wait(half - 2, 0)
        s_start(half - 1, 1)
        s_wait(half - 1, 1)

    plsc.subcore_barrier()

    out_rows = ACC_ROWS // NS
    pltpu.sync_copy(acc.at[pl.ds(s * out_rows, out_rows)],
                    out_hbm.at[c, pl.ds(s * out_rows, out_rows)])


@jax.jit
def _sc_aggregate(h, src2d, dst2d):
    mesh = plsc.VectorSubcoreMesh(core_axis_name="c", subcore_axis_name="s",
                                  num_cores=NC, num_subcores=NS)
    return pl.kernel(
        _agg_body,
        out_type=jax.ShapeDtypeStruct((NC, ACC_ROWS, F), jnp.float32),
        mesh=mesh,
        scratch_types=[
            pltpu.VMEM((NCH // 2, CHUNK), jnp.int32),
            pltpu.VMEM((NCH // 2, CHUNK), jnp.int32),
            pltpu.VMEM((CHUNK, F), jnp.float32),
            pltpu.VMEM((CHUNK, F), jnp.float32),
            pltpu.VMEM_SHARED((ACC_ROWS, F), jnp.float32),
            pltpu.SemaphoreType.DMA,
            pltpu.SemaphoreType.DMA,
            pltpu.SemaphoreType.DMA,
            pltpu.SemaphoreType.DMA,
        ],
    )(h, src2d, dst2d)


def _transform_body(h_ref, wc_ref, out_ref):
    out_ref[...] = jnp.dot(h_ref[...], wc_ref[...],
                           preferred_element_type=jnp.float32)


@jax.jit
def _tc_transform(h, wc):
    nb = 10
    rows = N_NODES // nb
    return pl.pallas_call(
        _transform_body,
        grid=(nb,),
        in_specs=[
            pl.BlockSpec((rows, F), lambda i: (i, 0)),
            pl.BlockSpec((F, F), lambda i: (0, 0)),
        ],
        out_specs=pl.BlockSpec((rows, F), lambda i: (i, 0)),
        out_shape=jax.ShapeDtypeStruct((N_NODES, F), jnp.float32),
    )(h, wc)


def _layer_tc_body(parts_ref, h_ref, lw_ref, b_ref, out_ref):
    acc = parts_ref[0] + parts_ref[1]
    acc += jnp.dot(h_ref[...], lw_ref[...], preferred_element_type=jnp.float32)
    acc += b_ref[...]
    out_ref[...] = jnp.maximum(acc, 0.0)


@jax.jit
def _tc_layer(parts, h, lw, b2d):
    nb = 10
    rows = N_NODES // nb
    return pl.pallas_call(
        _layer_tc_body,
        grid=(nb,),
        in_specs=[
            pl.BlockSpec((NC, rows, F), lambda i: (0, i, 0)),
            pl.BlockSpec((rows, F), lambda i: (i, 0)),
            pl.BlockSpec((F, F), lambda i: (0, 0)),
            pl.BlockSpec((1, F), lambda i: (0, 0)),
        ],
        out_specs=pl.BlockSpec((rows, F), lambda i: (i, 0)),
        out_shape=jax.ShapeDtypeStruct((N_NODES, F), jnp.float32),
    )(parts, h, lw, b2d)


def _pair_body(h_hbm, idx_hbm, out_hbm, idxv, bufa, bufb, sem):
    c = lax.axis_index("c")
    s = lax.axis_index("s")
    w = c * NS + s
    rows_per_w = N_PAIRS // CHUNK // NW

    pltpu.sync_copy(idx_hbm.at[w], idxv)

    for t in range(2):
        for k in range(rows_per_w):
            j = w * rows_per_w + k
            pltpu.async_copy(h_hbm.at[idxv.at[(2 * t) * 2 + k]], bufa, sem).wait()
            pltpu.async_copy(h_hbm.at[idxv.at[(2 * t + 1) * 2 + k]], bufb, sem).wait()

            def mrow(r, _):
                for cb in range(8):
                    sl = pl.ds(cb * 16, 16)
                    bufa[r, sl] = bufa[r, sl] * bufb[r, sl]
                return 0

            lax.fori_loop(0, CHUNK, mrow, 0)
            pltpu.sync_copy(bufa, out_hbm.at[t, pl.ds(j * CHUNK, CHUNK)])


@jax.jit
def _sc_pair_products(h, idx4):
    mesh = plsc.VectorSubcoreMesh(core_axis_name="c", subcore_axis_name="s",
                                  num_cores=NC, num_subcores=NS)
    rows_per_w = N_PAIRS // CHUNK // NW
    return pl.kernel(
        _pair_body,
        out_type=jax.ShapeDtypeStruct((2, N_PAIRS, F), jnp.float32),
        mesh=mesh,
        scratch_types=[
            pltpu.VMEM((4 * rows_per_w, CHUNK), jnp.int32),
            pltpu.VMEM((CHUNK, F), jnp.float32),
            pltpu.VMEM((CHUNK, F), jnp.float32),
            pltpu.SemaphoreType.DMA,
        ],
    )(h, idx4)


def _pred_body(x_ref, w1_ref, b1_ref, w2_ref, b2_ref, out_ref):
    z = jnp.dot(x_ref[...], w1_ref[...], preferred_element_type=jnp.float32)
    z = jnp.maximum(z + b1_ref[...], 0.0)
    o = jnp.dot(z, w2_ref[...], preferred_element_type=jnp.float32)
    out_ref[...] = o + b2_ref[...]


@jax.jit
def _tc_predictor(x, pW1, pb1, pW2, pb2):
    m = 2 * N_PAIRS
    nb = 16
    rows = m // nb
    return pl.pallas_call(
        _pred_body,
        grid=(nb,),
        in_specs=[
            pl.BlockSpec((rows, F), lambda i: (i, 0)),
            pl.BlockSpec((F, F), lambda i: (0, 0)),
            pl.BlockSpec((1, F), lambda i: (0, 0)),
            pl.BlockSpec((F, F), lambda i: (0, 0)),
            pl.BlockSpec((1, F), lambda i: (0, 0)),
        ],
        out_specs=pl.BlockSpec((rows, F), lambda i: (i, 0)),
        out_shape=jax.ShapeDtypeStruct((m, F), jnp.float32),
    )(x, pW1, pb1, pW2, pb2)


def _pad_edges(edge_index):
    src = edge_index[0].astype(jnp.int32)
    dst = edge_index[1].astype(jnp.int32)
    pad = E_PAD - N_EDGES
    src = jnp.concatenate([src, jnp.zeros((pad,), jnp.int32)])
    dst = jnp.concatenate([dst, jnp.full((pad,), DUMMY_ROW, jnp.int32)])
    return src.reshape(TOTAL_CH, CHUNK), dst.reshape(TOTAL_CH, CHUNK)


def _block_diag(W):
    nb, bi, bo = W.shape
    eye = jnp.eye(nb, dtype=W.dtype)
    return (eye[:, None, :, None] * W[:, :, None, :]).reshape(nb * bi, nb * bo)


def kernel(x, edge_index1, edge_index2, pos_src, pos_dst, neg_src, neg_dst,
           W1, loopW1, b1, W2, loopW2, b2, pW1, pb1, pW2, pb2):
    src1, dst1 = _pad_edges(edge_index1)
    src2, dst2 = _pad_edges(edge_index2)

    wc1 = _block_diag(W1)
    wc2 = _block_diag(W2)

    t1 = _tc_transform(x, wc1)
    parts1 = _sc_aggregate(t1, src1, dst1)
    h1 = _tc_layer(parts1, x, loopW1, b1.reshape(1, F))

    t2 = _tc_transform(h1, wc2)
    parts2 = _sc_aggregate(t2, src2, dst2)
    h2 = _tc_layer(parts2, h1, loopW2, b2.reshape(1, F))

    idx4 = jnp.stack([pos_src, pos_dst, neg_src, neg_dst]).astype(jnp.int32)
    idx4 = idx4.reshape(4, NW, N_PAIRS // CHUNK // NW, CHUNK)
    idx4 = idx4.transpose(1, 0, 2, 3).reshape(NW, -1, CHUNK)
    prods = _sc_pair_products(h2, idx4)

    hid = pW1.shape[1]
    w1p = jnp.zeros((F, F), jnp.float32).at[:, :hid].set(pW1)
    b1p = jnp.zeros((1, F), jnp.float32).at[:, :hid].set(pb1)
    w2p = jnp.zeros((F, F), jnp.float32).at[:hid, 0].set(pW2[:, 0])
    b2p = jnp.zeros((1, F), jnp.float32).at[:, 0].set(pb2[0])
    scores = _tc_predictor(prods.reshape(2 * N_PAIRS, F), w1p, b1p, w2p, b2p)
    h_pos = scores[:N_PAIRS, :1]
    h_neg = scores[N_PAIRS:, :1]
    return (h_pos, h_neg, h2)


# speedup vs baseline: 3.3441x; 1.0003x over previous
import functools
import jax
import jax.numpy as jnp
from jax import lax
from jax.experimental import pallas as pl
from jax.experimental.pallas import tpu as pltpu
from jax.experimental.pallas import tpu_sc as plsc

N_NODES = 10000
F = 128
N_EDGES = 320000
N_PAIRS = 8192

NC = 2
NS = 16
NW = NC * NS

CHUNK = 128
EDGES_PER_W = 10240
NCH = EDGES_PER_W // CHUNK
E_PAD = EDGES_PER_W * NW
TOTAL_CH = E_PAD // CHUNK
ACC_ROWS = 10240
DUMMY_ROW = N_NODES

_ZERO16 = None


def _agg_body(h_hbm, src_hbm, dst_hbm, out_hbm, sidx, didx, rows0, rows1, acc,
              sg0, sg1, ss0, ss1):
    c = lax.axis_index("c")
    s = lax.axis_index("s")
    w = c * NS + s

    zero16 = jnp.zeros((16,), jnp.float32)

    def zrow(r, _):
        for cb in range(8):
            rows0[r, pl.ds(cb * 16, 16)] = zero16
        return 0

    lax.fori_loop(0, CHUNK, zrow, 0)
    rows_per_tile = ACC_ROWS // NS
    for k in range(rows_per_tile // CHUNK):
        pltpu.sync_copy(rows0, acc.at[pl.ds(s * rows_per_tile + k * CHUNK, CHUNK)])

    plsc.subcore_barrier()

    half = NCH // 2
    bufs = (rows0, rows1)
    gsem = (sg0, sg1)
    ssem = (ss0, ss1)

    def g_start(j, b):
        pltpu.async_copy(h_hbm.at[sidx.at[j]], bufs[b], gsem[b])

    def g_wait(j, b):
        pltpu.make_async_copy(h_hbm.at[sidx.at[j]], bufs[b], gsem[b]).wait()

    def s_start(j, b):
        pltpu.async_copy(bufs[b], acc.at[didx.at[j]], ssem[b], add=True)

    def s_wait(j, b):
        pltpu.make_async_copy(bufs[b], acc.at[didx.at[j]], ssem[b]).wait()

    for hf in range(2):
        pltpu.sync_copy(src_hbm.at[pl.ds(w * NCH + hf * half, half)], sidx)
        pltpu.sync_copy(dst_hbm.at[pl.ds(w * NCH + hf * half, half)], didx)

        g_start(0, 0)
        g_wait(0, 0)
        s_start(0, 0)
        g_start(1, 1)

        def chunk_body(i, _):
            for p in range(2):
                j = 2 * i + 1 + p
                b = (1 - p)
                g_wait(j, b)
                s_start(j, b)
                s_wait(j - 1, 1 - b)
                g_start(j + 1, 1 - b)
            return 0

        lax.fori_loop(0, (half - 2) // 2, chunk_body, 0)

        g_wait(half - 1, 1)
        s_start(half - 1, 1)
        s_wait(half - 2, 0)
        s_wait(half - 1, 1)

    plsc.subcore_barrier()

    out_rows = ACC_ROWS // NS
    pltpu.sync_copy(acc.at[pl.ds(s * out_rows, out_rows)],
                    out_hbm.at[c, pl.ds(s * out_rows, out_rows)])


@jax.jit
def _sc_aggregate(h, src2d, dst2d):
    mesh = plsc.VectorSubcoreMesh(core_axis_name="c", subcore_axis_name="s",
                                  num_cores=NC, num_subcores=NS)
    return pl.kernel(
        _agg_body,
        out_type=jax.ShapeDtypeStruct((NC, ACC_ROWS, F), jnp.float32),
        mesh=mesh,
        scratch_types=[
            pltpu.VMEM((NCH // 2, CHUNK), jnp.int32),
            pltpu.VMEM((NCH // 2, CHUNK), jnp.int32),
            pltpu.VMEM((CHUNK, F), jnp.float32),
            pltpu.VMEM((CHUNK, F), jnp.float32),
            pltpu.VMEM_SHARED((ACC_ROWS, F), jnp.float32),
            pltpu.SemaphoreType.DMA,
            pltpu.SemaphoreType.DMA,
            pltpu.SemaphoreType.DMA,
            pltpu.SemaphoreType.DMA,
        ],
    )(h, src2d, dst2d)


def _transform_body(h_ref, wc_ref, out_ref):
    out_ref[...] = jnp.dot(h_ref[...], wc_ref[...],
                           preferred_element_type=jnp.float32)


@jax.jit
def _tc_transform(h, wc):
    nb = 10
    rows = N_NODES // nb
    return pl.pallas_call(
        _transform_body,
        grid=(nb,),
        in_specs=[
            pl.BlockSpec((rows, F), lambda i: (i, 0)),
            pl.BlockSpec((F, F), lambda i: (0, 0)),
        ],
        out_specs=pl.BlockSpec((rows, F), lambda i: (i, 0)),
        out_shape=jax.ShapeDtypeStruct((N_NODES, F), jnp.float32),
    )(h, wc)


def _layer_tc_body(parts_ref, h_ref, lw_ref, b_ref, out_ref):
    acc = parts_ref[0] + parts_ref[1]
    acc += jnp.dot(h_ref[...], lw_ref[...], preferred_element_type=jnp.float32)
    acc += b_ref[...]
    out_ref[...] = jnp.maximum(acc, 0.0)


@jax.jit
def _tc_layer(parts, h, lw, b2d):
    nb = 10
    rows = N_NODES // nb
    return pl.pallas_call(
        _layer_tc_body,
        grid=(nb,),
        in_specs=[
            pl.BlockSpec((NC, rows, F), lambda i: (0, i, 0)),
            pl.BlockSpec((rows, F), lambda i: (i, 0)),
            pl.BlockSpec((F, F), lambda i: (0, 0)),
            pl.BlockSpec((1, F), lambda i: (0, 0)),
        ],
        out_specs=pl.BlockSpec((rows, F), lambda i: (i, 0)),
        out_shape=jax.ShapeDtypeStruct((N_NODES, F), jnp.float32),
    )(parts, h, lw, b2d)


def _pair_body(h_hbm, idx_hbm, out_hbm, idxv, bufa, bufb, sem):
    c = lax.axis_index("c")
    s = lax.axis_index("s")
    w = c * NS + s
    rows_per_w = N_PAIRS // CHUNK // NW

    pltpu.sync_copy(idx_hbm.at[w], idxv)

    for t in range(2):
        for k in range(rows_per_w):
            j = w * rows_per_w + k
            pltpu.async_copy(h_hbm.at[idxv.at[(2 * t) * 2 + k]], bufa, sem).wait()
            pltpu.async_copy(h_hbm.at[idxv.at[(2 * t + 1) * 2 + k]], bufb, sem).wait()

            def mrow(r, _):
                for cb in range(8):
                    sl = pl.ds(cb * 16, 16)
                    bufa[r, sl] = bufa[r, sl] * bufb[r, sl]
                return 0

            lax.fori_loop(0, CHUNK, mrow, 0)
            pltpu.sync_copy(bufa, out_hbm.at[t, pl.ds(j * CHUNK, CHUNK)])


@jax.jit
def _sc_pair_products(h, idx4):
    mesh = plsc.VectorSubcoreMesh(core_axis_name="c", subcore_axis_name="s",
                                  num_cores=NC, num_subcores=NS)
    rows_per_w = N_PAIRS // CHUNK // NW
    return pl.kernel(
        _pair_body,
        out_type=jax.ShapeDtypeStruct((2, N_PAIRS, F), jnp.float32),
        mesh=mesh,
        scratch_types=[
            pltpu.VMEM((4 * rows_per_w, CHUNK), jnp.int32),
            pltpu.VMEM((CHUNK, F), jnp.float32),
            pltpu.VMEM((CHUNK, F), jnp.float32),
            pltpu.SemaphoreType.DMA,
        ],
    )(h, idx4)


def _pred_body(x_ref, w1_ref, b1_ref, w2_ref, b2_ref, out_ref):
    z = jnp.dot(x_ref[...], w1_ref[...], preferred_element_type=jnp.float32)
    z = jnp.maximum(z + b1_ref[...], 0.0)
    o = jnp.dot(z, w2_ref[...], preferred_element_type=jnp.float32)
    out_ref[...] = o + b2_ref[...]


@jax.jit
def _tc_predictor(x, pW1, pb1, pW2, pb2):
    m = 2 * N_PAIRS
    nb = 16
    rows = m // nb
    return pl.pallas_call(
        _pred_body,
        grid=(nb,),
        in_specs=[
            pl.BlockSpec((rows, F), lambda i: (i, 0)),
            pl.BlockSpec((F, F), lambda i: (0, 0)),
            pl.BlockSpec((1, F), lambda i: (0, 0)),
            pl.BlockSpec((F, F), lambda i: (0, 0)),
            pl.BlockSpec((1, F), lambda i: (0, 0)),
        ],
        out_specs=pl.BlockSpec((rows, F), lambda i: (i, 0)),
        out_shape=jax.ShapeDtypeStruct((m, F), jnp.float32),
    )(x, pW1, pb1, pW2, pb2)


def _pad_edges(edge_index):
    src = edge_index[0].astype(jnp.int32)
    dst = edge_index[1].astype(jnp.int32)
    pad = E_PAD - N_EDGES
    src = jnp.concatenate([src, jnp.zeros((pad,), jnp.int32)])
    dst = jnp.concatenate([dst, jnp.full((pad,), DUMMY_ROW, jnp.int32)])
    return src.reshape(TOTAL_CH, CHUNK), dst.reshape(TOTAL_CH, CHUNK)


def _block_diag(W):
    nb, bi, bo = W.shape
    eye = jnp.eye(nb, dtype=W.dtype)
    return (eye[:, None, :, None] * W[:, :, None, :]).reshape(nb * bi, nb * bo)


def kernel(x, edge_index1, edge_index2, pos_src, pos_dst, neg_src, neg_dst,
           W1, loopW1, b1, W2, loopW2, b2, pW1, pb1, pW2, pb2):
    src1, dst1 = _pad_edges(edge_index1)
    src2, dst2 = _pad_edges(edge_index2)

    wc1 = _block_diag(W1)
    wc2 = _block_diag(W2)

    t1 = _tc_transform(x, wc1)
    parts1 = _sc_aggregate(t1, src1, dst1)
    h1 = _tc_layer(parts1, x, loopW1, b1.reshape(1, F))

    t2 = _tc_transform(h1, wc2)
    parts2 = _sc_aggregate(t2, src2, dst2)
    h2 = _tc_layer(parts2, h1, loopW2, b2.reshape(1, F))

    idx4 = jnp.stack([pos_src, pos_dst, neg_src, neg_dst]).astype(jnp.int32)
    idx4 = idx4.reshape(4, NW, N_PAIRS // CHUNK // NW, CHUNK)
    idx4 = idx4.transpose(1, 0, 2, 3).reshape(NW, -1, CHUNK)
    prods = _sc_pair_products(h2, idx4)

    hid = pW1.shape[1]
    w1p = jnp.zeros((F, F), jnp.float32).at[:, :hid].set(pW1)
    b1p = jnp.zeros((1, F), jnp.float32).at[:, :hid].set(pb1)
    w2p = jnp.zeros((F, F), jnp.float32).at[:hid, 0].set(pW2[:, 0])
    b2p = jnp.zeros((1, F), jnp.float32).at[:, 0].set(pb2[0])
    scores = _tc_predictor(prods.reshape(2 * N_PAIRS, F), w1p, b1p, w2p, b2p)
    h_pos = scores[:N_PAIRS, :1]
    h_neg = scores[N_PAIRS:, :1]
    return (h_pos, h_neg, h2)


# speedup vs baseline: 3.5924x; 1.0742x over previous
import functools
import jax
import jax.numpy as jnp
from jax import lax
from jax.experimental import pallas as pl
from jax.experimental.pallas import tpu as pltpu
from jax.experimental.pallas import tpu_sc as plsc

N_NODES = 10000
F = 128
N_EDGES = 320000
N_PAIRS = 8192

NC = 2
NS = 16
NW = NC * NS

CHUNK = 128
EDGES_PER_W = 10240
NCH = EDGES_PER_W // CHUNK
E_PAD = EDGES_PER_W * NW
TOTAL_CH = E_PAD // CHUNK
ACC_ROWS = 10240
DUMMY_ROW = N_NODES

_ZERO16 = None


def _agg_body(h_hbm, src_hbm, dst_hbm, out_hbm, sidx, didx, rows0, rows1, acc,
              sg0, sg1, ss0, ss1):
    c = lax.axis_index("c")
    s = lax.axis_index("s")
    w = c * NS + s

    zero16 = jnp.zeros((16,), jnp.float32)

    def zrow(r, _):
        for cb in range(8):
            rows0[r, pl.ds(cb * 16, 16)] = zero16
        return 0

    lax.fori_loop(0, CHUNK, zrow, 0)
    rows_per_tile = ACC_ROWS // NS
    for k in range(rows_per_tile // CHUNK):
        pltpu.sync_copy(rows0, acc.at[pl.ds(s * rows_per_tile + k * CHUNK, CHUNK)])

    plsc.subcore_barrier()

    half = NCH // 2
    bufs = (rows0, rows1)
    gsem = (sg0, sg1)
    ssem = (ss0, ss1)

    def g_start(j, b):
        pltpu.async_copy(h_hbm.at[sidx.at[j]], bufs[b], gsem[b])

    def g_wait(j, b):
        pltpu.make_async_copy(h_hbm.at[sidx.at[j]], bufs[b], gsem[b]).wait()

    def s_start(j, b):
        pltpu.async_copy(bufs[b], acc.at[didx.at[j]], ssem[b], add=True)

    def s_wait(j, b):
        pltpu.make_async_copy(bufs[b], acc.at[didx.at[j]], ssem[b]).wait()

    for hf in range(2):
        pltpu.sync_copy(src_hbm.at[pl.ds(w * NCH + hf * half, half)], sidx)
        pltpu.sync_copy(dst_hbm.at[pl.ds(w * NCH + hf * half, half)], didx)

        g_start(0, 0)
        g_start(1, 1)

        def chunk_body(i, _):
            for p in range(2):
                j = 2 * i + p
                g_wait(j, p)
                s_start(j, p)
                s_wait(j, p)
                g_start(j + 2, p)
            return 0

        lax.fori_loop(0, (half - 2) // 2, chunk_body, 0)

        for p in range(2):
            j = half - 2 + p
            g_wait(j, p)
            s_start(j, p)
            s_wait(j, p)

    plsc.subcore_barrier()

    out_rows = ACC_ROWS // NS
    pltpu.sync_copy(acc.at[pl.ds(s * out_rows, out_rows)],
                    out_hbm.at[c, pl.ds(s * out_rows, out_rows)])


@jax.jit
def _sc_aggregate(h, src2d, dst2d):
    mesh = plsc.VectorSubcoreMesh(core_axis_name="c", subcore_axis_name="s",
                                  num_cores=NC, num_subcores=NS)
    return pl.kernel(
        _agg_body,
        out_type=jax.ShapeDtypeStruct((NC, ACC_ROWS, F), jnp.float32),
        mesh=mesh,
        scratch_types=[
            pltpu.VMEM((NCH // 2, CHUNK), jnp.int32),
            pltpu.VMEM((NCH // 2, CHUNK), jnp.int32),
            pltpu.VMEM((CHUNK, F), jnp.float32),
            pltpu.VMEM((CHUNK, F), jnp.float32),
            pltpu.VMEM_SHARED((ACC_ROWS, F), jnp.float32),
            pltpu.SemaphoreType.DMA,
            pltpu.SemaphoreType.DMA,
            pltpu.SemaphoreType.DMA,
            pltpu.SemaphoreType.DMA,
        ],
    )(h, src2d, dst2d)


def _transform_body(h_ref, wc_ref, out_ref):
    out_ref[...] = jnp.dot(h_ref[...], wc_ref[...],
                           preferred_element_type=jnp.float32)


@jax.jit
def _tc_transform(h, wc):
    nb = 10
    rows = N_NODES // nb
    return pl.pallas_call(
        _transform_body,
        grid=(nb,),
        in_specs=[
            pl.BlockSpec((rows, F), lambda i: (i, 0)),
            pl.BlockSpec((F, F), lambda i: (0, 0)),
        ],
        out_specs=pl.BlockSpec((rows, F), lambda i: (i, 0)),
        out_shape=jax.ShapeDtypeStruct((N_NODES, F), jnp.float32),
    )(h, wc)


def _layer_tc_body(parts_ref, h_ref, lw_ref, b_ref, out_ref):
    acc = parts_ref[0] + parts_ref[1]
    acc += jnp.dot(h_ref[...], lw_ref[...], preferred_element_type=jnp.float32)
    acc += b_ref[...]
    out_ref[...] = jnp.maximum(acc, 0.0)


@jax.jit
def _tc_layer(parts, h, lw, b2d):
    nb = 10
    rows = N_NODES // nb
    return pl.pallas_call(
        _layer_tc_body,
        grid=(nb,),
        in_specs=[
            pl.BlockSpec((NC, rows, F), lambda i: (0, i, 0)),
            pl.BlockSpec((rows, F), lambda i: (i, 0)),
            pl.BlockSpec((F, F), lambda i: (0, 0)),
            pl.BlockSpec((1, F), lambda i: (0, 0)),
        ],
        out_specs=pl.BlockSpec((rows, F), lambda i: (i, 0)),
        out_shape=jax.ShapeDtypeStruct((N_NODES, F), jnp.float32),
    )(parts, h, lw, b2d)


def _pair_body(h_hbm, idx_hbm, out_hbm, idxv, bufa, bufb, sem):
    c = lax.axis_index("c")
    s = lax.axis_index("s")
    w = c * NS + s
    rows_per_w = N_PAIRS // CHUNK // NW

    pltpu.sync_copy(idx_hbm.at[w], idxv)

    for t in range(2):
        for k in range(rows_per_w):
            j = w * rows_per_w + k
            pltpu.async_copy(h_hbm.at[idxv.at[(2 * t) * 2 + k]], bufa, sem).wait()
            pltpu.async_copy(h_hbm.at[idxv.at[(2 * t + 1) * 2 + k]], bufb, sem).wait()

            def mrow(r, _):
                for cb in range(8):
                    sl = pl.ds(cb * 16, 16)
                    bufa[r, sl] = bufa[r, sl] * bufb[r, sl]
                return 0

            lax.fori_loop(0, CHUNK, mrow, 0)
            pltpu.sync_copy(bufa, out_hbm.at[t, pl.ds(j * CHUNK, CHUNK)])


@jax.jit
def _sc_pair_products(h, idx4):
    mesh = plsc.VectorSubcoreMesh(core_axis_name="c", subcore_axis_name="s",
                                  num_cores=NC, num_subcores=NS)
    rows_per_w = N_PAIRS // CHUNK // NW
    return pl.kernel(
        _pair_body,
        out_type=jax.ShapeDtypeStruct((2, N_PAIRS, F), jnp.float32),
        mesh=mesh,
        scratch_types=[
            pltpu.VMEM((4 * rows_per_w, CHUNK), jnp.int32),
            pltpu.VMEM((CHUNK, F), jnp.float32),
            pltpu.VMEM((CHUNK, F), jnp.float32),
            pltpu.SemaphoreType.DMA,
        ],
    )(h, idx4)


def _pred_body(x_ref, w1_ref, b1_ref, w2_ref, b2_ref, out_ref):
    z = jnp.dot(x_ref[...], w1_ref[...], preferred_element_type=jnp.float32)
    z = jnp.maximum(z + b1_ref[...], 0.0)
    o = jnp.dot(z, w2_ref[...], preferred_element_type=jnp.float32)
    out_ref[...] = o + b2_ref[...]


@jax.jit
def _tc_predictor(x, pW1, pb1, pW2, pb2):
    m = 2 * N_PAIRS
    nb = 16
    rows = m // nb
    return pl.pallas_call(
        _pred_body,
        grid=(nb,),
        in_specs=[
            pl.BlockSpec((rows, F), lambda i: (i, 0)),
            pl.BlockSpec((F, F), lambda i: (0, 0)),
            pl.BlockSpec((1, F), lambda i: (0, 0)),
            pl.BlockSpec((F, F), lambda i: (0, 0)),
            pl.BlockSpec((1, F), lambda i: (0, 0)),
        ],
        out_specs=pl.BlockSpec((rows, F), lambda i: (i, 0)),
        out_shape=jax.ShapeDtypeStruct((m, F), jnp.float32),
    )(x, pW1, pb1, pW2, pb2)


def _pad_edges(edge_index):
    src = edge_index[0].astype(jnp.int32)
    dst = edge_index[1].astype(jnp.int32)
    pad = E_PAD - N_EDGES
    src = jnp.concatenate([src, jnp.zeros((pad,), jnp.int32)])
    dst = jnp.concatenate([dst, jnp.full((pad,), DUMMY_ROW, jnp.int32)])
    return src.reshape(TOTAL_CH, CHUNK), dst.reshape(TOTAL_CH, CHUNK)


def _block_diag(W):
    nb, bi, bo = W.shape
    eye = jnp.eye(nb, dtype=W.dtype)
    return (eye[:, None, :, None] * W[:, :, None, :]).reshape(nb * bi, nb * bo)


def kernel(x, edge_index1, edge_index2, pos_src, pos_dst, neg_src, neg_dst,
           W1, loopW1, b1, W2, loopW2, b2, pW1, pb1, pW2, pb2):
    src1, dst1 = _pad_edges(edge_index1)
    src2, dst2 = _pad_edges(edge_index2)

    wc1 = _block_diag(W1)
    wc2 = _block_diag(W2)

    t1 = _tc_transform(x, wc1)
    parts1 = _sc_aggregate(t1, src1, dst1)
    h1 = _tc_layer(parts1, x, loopW1, b1.reshape(1, F))

    t2 = _tc_transform(h1, wc2)
    parts2 = _sc_aggregate(t2, src2, dst2)
    h2 = _tc_layer(parts2, h1, loopW2, b2.reshape(1, F))

    idx4 = jnp.stack([pos_src, pos_dst, neg_src, neg_dst]).astype(jnp.int32)
    idx4 = idx4.reshape(4, NW, N_PAIRS // CHUNK // NW, CHUNK)
    idx4 = idx4.transpose(1, 0, 2, 3).reshape(NW, -1, CHUNK)
    prods = _sc_pair_products(h2, idx4)

    hid = pW1.shape[1]
    w1p = jnp.zeros((F, F), jnp.float32).at[:, :hid].set(pW1)
    b1p = jnp.zeros((1, F), jnp.float32).at[:, :hid].set(pb1)
    w2p = jnp.zeros((F, F), jnp.float32).at[:hid, 0].set(pW2[:, 0])
    b2p = jnp.zeros((1, F), jnp.float32).at[:, 0].set(pb2[0])
    scores = _tc_predictor(prods.reshape(2 * N_PAIRS, F), w1p, b1p, w2p, b2p)
    h_pos = scores[:N_PAIRS, :1]
    h_neg = scores[N_PAIRS:, :1]
    return (h_pos, h_neg, h2)
